# Initial kernel scaffold; baseline (speedup 1.0000x reference)
import jax
import jax.numpy as jnp
from jax.experimental import pallas as pl


def kernel(x, adj, W_l, W_r, att, bias):
    raise NotImplementedError("write your pallas kernel here")


# speedup vs baseline: 685.1488x; 685.1488x over previous
import jax
import jax.numpy as jnp
from jax.experimental import pallas as pl

_B, _N, _IN, _OUT, _H = 4, 512, 128, 128, 4
_C = _OUT // _H
_CH = 32
_NCH = _N // _CH


def _gat_body(xT_ref, adjT_ref, wlT_ref, wrT_ref, attv_ref, bias_ref, out_ref):
    xT = xT_ref[0]
    wlT = wlT_ref[...]
    wrT = wrT_ref[...]
    attv = attv_ref[...]
    bias = bias_ref[...]

    xlT = jnp.dot(wlT, xT, preferred_element_type=jnp.float32)
    xrT = jnp.dot(wrT, xT, preferred_element_type=jnp.float32)

    for ci in range(_NCH):
        cs = ci * _CH
        xr_c = xrT[:, cs:cs + _CH]
        t = xlT[:, None, :] + xr_c[:, :, None]
        lz = jnp.maximum(t, 0.2 * t)
        u = lz * attv[:, :, None]
        logits = u.reshape(_H, _C, _CH, _N).sum(axis=1)

        adj_c = adjT_ref[0, cs:cs + _CH, :]
        jidx = jax.lax.broadcasted_iota(jnp.int32, (_CH, _N), 1)
        iidx = jax.lax.broadcasted_iota(jnp.int32, (_CH, _N), 0) + cs
        valid = (jidx == iidx) | (adj_c != 0)
        validb = valid[None, :, :]

        masked = jnp.where(validb, logits, -1e30)
        m = jnp.max(masked, axis=2, keepdims=True)
        ex = jnp.where(validb, jnp.exp(logits - m), 0.0)
        denom = jnp.sum(ex, axis=2, keepdims=True)
        alpha = ex / (denom + 1e-16)

        outs = []
        for h in range(_H):
            a_h = alpha[h]
            xl_h = xlT[h * _C:(h + 1) * _C, :]
            outs.append(jax.lax.dot_general(
                a_h, xl_h, (((1,), (1,)), ((), ())),
                preferred_element_type=jnp.float32))
        o = jnp.concatenate(outs, axis=1) + bias
        out_ref[0, cs:cs + _CH, :] = o


def kernel(x, adj, W_l, W_r, att, bias):
    xT = jnp.swapaxes(x, 1, 2)
    adjT = jnp.swapaxes(adj, 1, 2)
    wlT = W_l.T
    wrT = W_r.T
    attv = att.reshape(_OUT, 1)
    bias2 = bias.reshape(1, _OUT)

    return pl.pallas_call(
        _gat_body,
        grid=(_B,),
        in_specs=[
            pl.BlockSpec((1, _IN, _N), lambda b: (b, 0, 0)),
            pl.BlockSpec((1, _N, _N), lambda b: (b, 0, 0)),
            pl.BlockSpec((_OUT, _IN), lambda b: (0, 0)),
            pl.BlockSpec((_OUT, _IN), lambda b: (0, 0)),
            pl.BlockSpec((_OUT, 1), lambda b: (0, 0)),
            pl.BlockSpec((1, _OUT), lambda b: (0, 0)),
        ],
        out_specs=pl.BlockSpec((1, _N, _OUT), lambda b: (b, 0, 0)),
        out_shape=jax.ShapeDtypeStruct((_B, _N, _OUT), jnp.float32),
    )(xT, adjT, wlT, wrT, attv, bias2)


# speedup vs baseline: 773.0550x; 1.1283x over previous
import jax
import jax.numpy as jnp
from jax.experimental import pallas as pl

_B, _N, _IN, _OUT, _H = 4, 512, 128, 128, 4
_C = _OUT // _H
_CH = 32
_NCH = _N // _CH


def _gat_body(xT_ref, adjT_ref, wlT_ref, wrT_ref, attv_ref, qv_ref, bias_ref,
              out_ref):
    xT = xT_ref[0]
    wlT = wlT_ref[...]
    wrT = wrT_ref[...]
    attv = attv_ref[...]
    qv = qv_ref[...]
    bias = bias_ref[...]

    xlT = jnp.dot(wlT, xT, preferred_element_type=jnp.float32)
    xrT = jnp.dot(wrT, xT, preferred_element_type=jnp.float32)

    zl = xlT * attv
    zr = xrT * attv
    SL = 0.6 * zl.reshape(_H, _C, _N).sum(axis=1)
    SR = 0.6 * zr.reshape(_H, _C, _N).sum(axis=1)

    for ci in range(_NCH):
        cs = ci * _CH
        zr_c = zr[:, cs:cs + _CH]
        s = zl[:, None, :] + zr_c[:, :, None]
        w = qv[:, :, None] * jnp.abs(s)
        habs = w.reshape(_H, _C, _CH, _N).sum(axis=1)
        logits = habs + SL[:, None, :] + SR[:, cs:cs + _CH, None]

        adj_c = adjT_ref[0, cs:cs + _CH, :]
        jidx = jax.lax.broadcasted_iota(jnp.int32, (_CH, _N), 1)
        iidx = jax.lax.broadcasted_iota(jnp.int32, (_CH, _N), 0) + cs
        valid = (jidx == iidx) | (adj_c != 0)
        validb = valid[None, :, :]

        masked = jnp.where(validb, logits, -1e30)
        m = jnp.max(masked, axis=2, keepdims=True)
        ex = jnp.where(validb, jnp.exp(logits - m), 0.0)
        denom = jnp.sum(ex, axis=2, keepdims=True)
        alpha = ex / (denom + 1e-16)

        outs = []
        for h in range(_H):
            a_h = alpha[h]
            xl_h = xlT[h * _C:(h + 1) * _C, :]
            outs.append(jax.lax.dot_general(
                a_h, xl_h, (((1,), (1,)), ((), ())),
                preferred_element_type=jnp.float32))
        o = jnp.concatenate(outs, axis=1) + bias
        out_ref[0, cs:cs + _CH, :] = o


def kernel(x, adj, W_l, W_r, att, bias):
    xT = jnp.swapaxes(x, 1, 2)
    adjT = jnp.swapaxes(adj, 1, 2)
    wlT = W_l.T
    wrT = W_r.T
    attv = att.reshape(_OUT, 1)
    qv = 0.4 * jnp.sign(attv)
    bias2 = bias.reshape(1, _OUT)

    return pl.pallas_call(
        _gat_body,
        grid=(_B,),
        in_specs=[
            pl.BlockSpec((1, _IN, _N), lambda b: (b, 0, 0)),
            pl.BlockSpec((1, _N, _N), lambda b: (b, 0, 0)),
            pl.BlockSpec((_OUT, _IN), lambda b: (0, 0)),
            pl.BlockSpec((_OUT, _IN), lambda b: (0, 0)),
            pl.BlockSpec((_OUT, 1), lambda b: (0, 0)),
            pl.BlockSpec((_OUT, 1), lambda b: (0, 0)),
            pl.BlockSpec((1, _OUT), lambda b: (0, 0)),
        ],
        out_specs=pl.BlockSpec((1, _N, _OUT), lambda b: (b, 0, 0)),
        out_shape=jax.ShapeDtypeStruct((_B, _N, _OUT), jnp.float32),
    )(xT, adjT, wlT, wrT, attv, qv, bias2)
